# baseline (speedup 1.0000x reference)
import functools

import jax
import jax.numpy as jnp
from jax import lax
from jax.experimental import pallas as pl
from jax.experimental.pallas import tpu as pltpu
from jax.experimental.pallas import tpu_sc as plsc

_NC, _NS = 2, 16
_NW = _NC * _NS
_CH = 200
_D_NODE = 128
_BE = 2560


_NRING = 2


def _gather_body(nodes_hbm, s_hbm, r_hbm, out_s, out_r,
                 idx_s, idx_r, bufs, sems_g, sems_w):
    e_total = s_hbm.shape[0]
    b_per_w = e_total // _NW
    nch = b_per_w // _CH
    wid = lax.axis_index("s") * _NC + lax.axis_index("c")
    base = wid * b_per_w

    pltpu.sync_copy(s_hbm.at[pl.ds(base, b_per_w)], idx_s)
    pltpu.sync_copy(r_hbm.at[pl.ds(base, b_per_w)], idx_r)
    idx = (idx_s, idx_r)
    out = (out_s, out_r)

    def fire_gather(ring, c):
        return [
            pltpu.async_copy(
                nodes_hbm.at[idx[st].at[pl.ds(c * _CH, _CH)]],
                bufs[ring][st], sems_g[ring][st])
            for st in (0, 1)
        ]

    def fire_write(ring, c):
        off = pl.multiple_of(base + c * _CH, 8)
        for st in (0, 1):
            pltpu.async_copy(bufs[ring][st], out[st].at[pl.ds(off, _CH)],
                             sems_w[ring][st])

    def wait_write(ring):
        for st in (0, 1):
            pltpu.make_async_copy(bufs[ring][st],
                                  out[st].at[pl.ds(0, _CH)],
                                  sems_w[ring][st]).wait()

    ga = [fire_gather(j, j) for j in range(_NRING)]
    for j in range(_NRING):
        for d in ga[j]:
            d.wait()
        fire_write(j, j)

    def body(t, carry):
        c0 = t * _NRING
        ga = []
        for j in range(_NRING):
            wait_write(j)
            ga.append(fire_gather(j, c0 + j))
        for j in range(_NRING):
            for d in ga[j]:
                d.wait()
            fire_write(j, c0 + j)
        return carry

    lax.fori_loop(1, nch // _NRING, body, 0)
    for j in range(_NRING):
        wait_write(j)


def _sc_gather(nodes, senders, receivers):
    e_total = senders.shape[0]
    b_per_w = e_total // _NW
    nw = nodes.shape[1]
    mesh = plsc.VectorSubcoreMesh(core_axis_name="c", subcore_axis_name="s")
    f = pl.kernel(
        _gather_body,
        out_type=[
            jax.ShapeDtypeStruct((e_total, nw), jnp.int32),
            jax.ShapeDtypeStruct((e_total, nw), jnp.int32),
        ],
        mesh=mesh,
        compiler_params=pltpu.CompilerParams(use_tc_tiling_on_sc=False),
        scratch_types=[
            pltpu.VMEM((b_per_w,), jnp.int32),
            pltpu.VMEM((b_per_w,), jnp.int32),
            [[pltpu.VMEM((_CH, nw), jnp.int32) for _ in range(2)]
             for _ in range(_NRING)],
            [[pltpu.SemaphoreType.DMA for _ in range(2)] for _ in range(_NRING)],
            [[pltpu.SemaphoreType.DMA for _ in range(2)] for _ in range(_NRING)],
        ],
    )
    return f(nodes, senders, receivers)


def _mlp_body(sg_ref, rg_ref, ef_ref, w1a_ref, w1b_ref, w1c_ref, b1_ref,
              w2_ref, b2_ref, out_ref):
    x = (jnp.dot(sg_ref[...], w1a_ref[...], preferred_element_type=jnp.float32)
         + jnp.dot(rg_ref[...], w1b_ref[...], preferred_element_type=jnp.float32)
         + jnp.dot(ef_ref[...], w1c_ref[...], preferred_element_type=jnp.float32)
         + b1_ref[...])
    h = 1.0 / (1.0 + jnp.exp(-x))
    out_ref[...] = (jnp.dot(h, w2_ref[...], preferred_element_type=jnp.float32)
                    + b2_ref[...])


def _tc_mlp(sg, rg, edges, w1a, w1b, w1c, b1, w2, b2):
    e_total, d_edge = edges.shape
    d_hidden = w2.shape[0]
    d_out = w2.shape[1]
    grid = (e_total // _BE,)
    return pl.pallas_call(
        _mlp_body,
        grid=grid,
        in_specs=[
            pl.BlockSpec((_BE, _D_NODE), lambda i: (i, 0)),
            pl.BlockSpec((_BE, _D_NODE), lambda i: (i, 0)),
            pl.BlockSpec((_BE, d_edge), lambda i: (i, 0)),
            pl.BlockSpec((_D_NODE, d_hidden), lambda i: (0, 0)),
            pl.BlockSpec((_D_NODE, d_hidden), lambda i: (0, 0)),
            pl.BlockSpec((d_edge, d_hidden), lambda i: (0, 0)),
            pl.BlockSpec((1, d_hidden), lambda i: (0, 0)),
            pl.BlockSpec((d_hidden, d_out), lambda i: (0, 0)),
            pl.BlockSpec((1, d_out), lambda i: (0, 0)),
        ],
        out_specs=pl.BlockSpec((_BE, d_out), lambda i: (i, 0)),
        out_shape=jax.ShapeDtypeStruct((e_total, d_out), jnp.float32),
    )(sg, rg, edges, w1a, w1b, w1c, b1, w2, b2)


_NCHUNK = 5


@jax.jit
def kernel(nodes, edges, W1, b1, W2, b2, senders, receivers):
    senders = senders.astype(jnp.int32)
    receivers = receivers.astype(jnp.int32)
    e_total = senders.shape[0]
    ec = e_total // _NCHUNK
    assert ec % (_NW * _NRING * _CH) == 0 and ec % _BE == 0

    d_node = nodes.shape[1]
    n_nodes = nodes.shape[0]
    nodes_bf = nodes.astype(jnp.bfloat16)
    nodes_pk = lax.bitcast_convert_type(
        nodes_bf.reshape(n_nodes, d_node // 2, 2), jnp.int32)
    w1t = W1.T
    w1a = w1t[:d_node].astype(jnp.bfloat16)
    w1b = w1t[d_node:2 * d_node].astype(jnp.bfloat16)
    w1c = w1t[2 * d_node:]
    b1r = b1.reshape(1, -1)
    w2t = W2.T
    b2r = b2.reshape(1, -1)

    def unpack(x):
        return lax.bitcast_convert_type(x, jnp.bfloat16).reshape(-1, d_node)

    gathered = []
    for k in range(_NCHUNK):
        sl = slice(k * ec, (k + 1) * ec)
        gathered.append(_sc_gather(nodes_pk, senders[sl], receivers[sl]))
    outs = []
    for k in range(_NCHUNK):
        sl = slice(k * ec, (k + 1) * ec)
        sg, rg = gathered[k]
        outs.append(_tc_mlp(unpack(sg), unpack(rg), edges[sl],
                            w1a, w1b, w1c, b1r, w2t, b2r))
    return jnp.concatenate(outs, axis=0)


# speedup vs baseline: 3.2744x; 3.2744x over previous
import functools

import jax
import jax.numpy as jnp
from jax import lax
from jax.experimental import pallas as pl
from jax.experimental.pallas import tpu as pltpu
from jax.experimental.pallas import tpu_sc as plsc

_NC, _NS = 2, 16
_NW = _NC * _NS
_CH = 200
_NRING = 2
_D_NODE = 128
_BE = 2560
_NCHUNK = 5


def _gather_body(nodes_hbm, s_hbm, r_hbm, out, idx_s, idx_r,
                 bufs, sems_g, sems_w):
    e_total = s_hbm.shape[0]
    b_per_w = e_total // _NW
    nch = b_per_w // _CH
    wid = lax.axis_index("s") * _NC + lax.axis_index("c")
    base = wid * b_per_w

    pltpu.sync_copy(s_hbm.at[pl.ds(base, b_per_w)], idx_s)
    pltpu.sync_copy(r_hbm.at[pl.ds(base, b_per_w)], idx_r)
    idx = (idx_s, idx_r)

    def fire_gather(ring, c):
        return [
            pltpu.async_copy(
                nodes_hbm.at[idx[st].at[pl.ds(c * _CH, _CH)]],
                bufs[ring][st], sems_g[ring][st])
            for st in (0, 1)
        ]

    def fire_write(ring, c):
        off = pl.multiple_of(base + c * _CH, 8)
        for st in (0, 1):
            pltpu.async_copy(
                bufs[ring][st],
                out.at[pl.ds(off, _CH), pl.ds(st * _D_NODE, _D_NODE)],
                sems_w[ring][st])

    def wait_write(ring):
        for st in (0, 1):
            pltpu.make_async_copy(
                bufs[ring][st],
                out.at[pl.ds(0, _CH), pl.ds(st * _D_NODE, _D_NODE)],
                sems_w[ring][st]).wait()

    ga = [fire_gather(j, j) for j in range(_NRING)]
    for j in range(_NRING):
        for d in ga[j]:
            d.wait()
        fire_write(j, j)

    def body(t, carry):
        c0 = t * _NRING
        ga = []
        for j in range(_NRING):
            wait_write(j)
            ga.append(fire_gather(j, c0 + j))
        for j in range(_NRING):
            for d in ga[j]:
                d.wait()
            fire_write(j, c0 + j)
        return carry

    lax.fori_loop(1, nch // _NRING, body, 0)
    for j in range(_NRING):
        wait_write(j)


def _sc_gather(nodes, senders, receivers):
    e_total = senders.shape[0]
    b_per_w = e_total // _NW
    mesh = plsc.VectorSubcoreMesh(core_axis_name="c", subcore_axis_name="s")
    f = pl.kernel(
        _gather_body,
        out_type=jax.ShapeDtypeStruct((e_total, 2 * _D_NODE), jnp.float32),
        mesh=mesh,
        scratch_types=[
            pltpu.VMEM((b_per_w,), jnp.int32),
            pltpu.VMEM((b_per_w,), jnp.int32),
            [[pltpu.VMEM((_CH, _D_NODE), jnp.float32) for _ in range(2)]
             for _ in range(_NRING)],
            [[pltpu.SemaphoreType.DMA for _ in range(2)] for _ in range(_NRING)],
            [[pltpu.SemaphoreType.DMA for _ in range(2)] for _ in range(_NRING)],
        ],
    )
    return f(nodes, senders, receivers)


def _mlp_body(g_ref, ef_ref, w1ab_ref, w1c_ref, b1_ref, w2_ref, b2_ref,
              out_ref):
    x = (jnp.dot(g_ref[...].astype(jnp.bfloat16), w1ab_ref[...],
                 preferred_element_type=jnp.float32)
         + jnp.dot(ef_ref[...].astype(jnp.bfloat16), w1c_ref[...],
                   preferred_element_type=jnp.float32)
         + b1_ref[...])
    h = 1.0 / (1.0 + jnp.exp(-x))
    out_ref[...] = (jnp.dot(h.astype(jnp.bfloat16), w2_ref[...],
                            preferred_element_type=jnp.float32)
                    + b2_ref[...])


def _tc_mlp(g, edges, w1ab, w1c, b1, w2, b2):
    e_total, d_edge = edges.shape
    d_hidden = w2.shape[0]
    d_out = w2.shape[1]
    grid = (e_total // _BE,)
    return pl.pallas_call(
        _mlp_body,
        grid=grid,
        in_specs=[
            pl.BlockSpec((_BE, 2 * _D_NODE), lambda i: (i, 0)),
            pl.BlockSpec((_BE, d_edge), lambda i: (i, 0)),
            pl.BlockSpec((2 * _D_NODE, d_hidden), lambda i: (0, 0)),
            pl.BlockSpec((d_edge, d_hidden), lambda i: (0, 0)),
            pl.BlockSpec((1, d_hidden), lambda i: (0, 0)),
            pl.BlockSpec((d_hidden, d_out), lambda i: (0, 0)),
            pl.BlockSpec((1, d_out), lambda i: (0, 0)),
        ],
        out_specs=pl.BlockSpec((_BE, d_out), lambda i: (i, 0)),
        out_shape=jax.ShapeDtypeStruct((e_total, d_out), jnp.float32),
    )(g, edges, w1ab, w1c, b1, w2, b2)


@jax.jit
def kernel(nodes, edges, W1, b1, W2, b2, senders, receivers):
    senders = senders.astype(jnp.int32)
    receivers = receivers.astype(jnp.int32)
    e_total = senders.shape[0]
    ec = e_total // _NCHUNK
    assert ec % (_NW * _NRING * _CH) == 0 and ec % _BE == 0

    d_node = nodes.shape[1]
    w1t = W1.T
    w1ab = w1t[:2 * d_node].astype(jnp.bfloat16)
    w1c = w1t[2 * d_node:].astype(jnp.bfloat16)
    b1r = b1.reshape(1, -1)
    w2t = W2.T.astype(jnp.bfloat16)
    b2r = b2.reshape(1, -1)

    gathered = []
    for k in range(_NCHUNK):
        sl = slice(k * ec, (k + 1) * ec)
        gathered.append(_sc_gather(nodes, senders[sl], receivers[sl]))
    outs = []
    for k in range(_NCHUNK):
        sl = slice(k * ec, (k + 1) * ec)
        outs.append(_tc_mlp(gathered[k], edges[sl],
                            w1ab, w1c, b1r, w2t, b2r))
    return jnp.concatenate(outs, axis=0)


# speedup vs baseline: 3.3137x; 1.0120x over previous
import functools

import jax
import jax.numpy as jnp
from jax import lax
from jax.experimental import pallas as pl
from jax.experimental.pallas import tpu as pltpu
from jax.experimental.pallas import tpu_sc as plsc

_NC, _NS = 2, 16
_NW = _NC * _NS
_CH = 200
_NRING = 2
_D_NODE = 128
_BE = 2560
_NCHUNK = 1


def _gather_body(nodes_hbm, s_hbm, r_hbm, out, idx_s, idx_r,
                 bufs, sems_g, sems_w):
    e_total = s_hbm.shape[0]
    b_per_w = e_total // _NW
    nch = b_per_w // _CH
    wid = lax.axis_index("s") * _NC + lax.axis_index("c")
    base = wid * b_per_w

    pltpu.sync_copy(s_hbm.at[pl.ds(base, b_per_w)], idx_s)
    pltpu.sync_copy(r_hbm.at[pl.ds(base, b_per_w)], idx_r)
    idx = (idx_s, idx_r)

    def fire_gather(ring, c):
        return [
            pltpu.async_copy(
                nodes_hbm.at[idx[st].at[pl.ds(c * _CH, _CH)]],
                bufs[ring][st], sems_g[ring][st])
            for st in (0, 1)
        ]

    def fire_write(ring, c):
        off = pl.multiple_of(base + c * _CH, 8)
        for st in (0, 1):
            pltpu.async_copy(
                bufs[ring][st],
                out.at[pl.ds(off, _CH), pl.ds(st * _D_NODE, _D_NODE)],
                sems_w[ring][st])

    def wait_write(ring):
        for st in (0, 1):
            pltpu.make_async_copy(
                bufs[ring][st],
                out.at[pl.ds(0, _CH), pl.ds(st * _D_NODE, _D_NODE)],
                sems_w[ring][st]).wait()

    ga = [fire_gather(j, j) for j in range(_NRING)]
    for j in range(_NRING):
        for d in ga[j]:
            d.wait()
        fire_write(j, j)

    def body(t, carry):
        c0 = t * _NRING
        ga = []
        for j in range(_NRING):
            wait_write(j)
            ga.append(fire_gather(j, c0 + j))
        for j in range(_NRING):
            for d in ga[j]:
                d.wait()
            fire_write(j, c0 + j)
        return carry

    lax.fori_loop(1, nch // _NRING, body, 0)
    for j in range(_NRING):
        wait_write(j)


def _sc_gather(nodes, senders, receivers):
    e_total = senders.shape[0]
    b_per_w = e_total // _NW
    mesh = plsc.VectorSubcoreMesh(core_axis_name="c", subcore_axis_name="s")
    f = pl.kernel(
        _gather_body,
        out_type=jax.ShapeDtypeStruct((e_total, 2 * _D_NODE), jnp.float32),
        mesh=mesh,
        scratch_types=[
            pltpu.VMEM((b_per_w,), jnp.int32),
            pltpu.VMEM((b_per_w,), jnp.int32),
            [[pltpu.VMEM((_CH, _D_NODE), jnp.float32) for _ in range(2)]
             for _ in range(_NRING)],
            [[pltpu.SemaphoreType.DMA for _ in range(2)] for _ in range(_NRING)],
            [[pltpu.SemaphoreType.DMA for _ in range(2)] for _ in range(_NRING)],
        ],
    )
    return f(nodes, senders, receivers)


def _mlp_body(g_ref, ef_ref, w1ab_ref, w1c_ref, b1_ref, w2_ref, b2_ref,
              out_ref):
    x = (jnp.dot(g_ref[...].astype(jnp.bfloat16), w1ab_ref[...],
                 preferred_element_type=jnp.float32)
         + jnp.dot(ef_ref[...].astype(jnp.bfloat16), w1c_ref[...],
                   preferred_element_type=jnp.float32)
         + b1_ref[...])
    h = 0.5 * jnp.tanh(0.5 * x) + 0.5
    out_ref[...] = (jnp.dot(h.astype(jnp.bfloat16), w2_ref[...],
                            preferred_element_type=jnp.float32)
                    + b2_ref[...])


def _tc_mlp(g, edges, w1ab, w1c, b1, w2, b2):
    e_total, d_edge = edges.shape
    d_hidden = w2.shape[0]
    d_out = w2.shape[1]
    grid = (e_total // _BE,)
    return pl.pallas_call(
        _mlp_body,
        grid=grid,
        in_specs=[
            pl.BlockSpec((_BE, 2 * _D_NODE), lambda i: (i, 0)),
            pl.BlockSpec((_BE, d_edge), lambda i: (i, 0)),
            pl.BlockSpec((2 * _D_NODE, d_hidden), lambda i: (0, 0)),
            pl.BlockSpec((d_edge, d_hidden), lambda i: (0, 0)),
            pl.BlockSpec((1, d_hidden), lambda i: (0, 0)),
            pl.BlockSpec((d_hidden, d_out), lambda i: (0, 0)),
            pl.BlockSpec((1, d_out), lambda i: (0, 0)),
        ],
        out_specs=pl.BlockSpec((_BE, d_out), lambda i: (i, 0)),
        out_shape=jax.ShapeDtypeStruct((e_total, d_out), jnp.float32),
    )(g, edges, w1ab, w1c, b1, w2, b2)


@jax.jit
def kernel(nodes, edges, W1, b1, W2, b2, senders, receivers):
    senders = senders.astype(jnp.int32)
    receivers = receivers.astype(jnp.int32)
    e_total = senders.shape[0]
    ec = e_total // _NCHUNK
    assert ec % (_NW * _NRING * _CH) == 0 and ec % _BE == 0

    d_node = nodes.shape[1]
    w1t = W1.T
    w1ab = w1t[:2 * d_node].astype(jnp.bfloat16)
    w1c = w1t[2 * d_node:].astype(jnp.bfloat16)
    b1r = b1.reshape(1, -1)
    w2t = W2.T.astype(jnp.bfloat16)
    b2r = b2.reshape(1, -1)

    gathered = []
    for k in range(_NCHUNK):
        sl = slice(k * ec, (k + 1) * ec)
        gathered.append(_sc_gather(nodes, senders[sl], receivers[sl]))
    outs = []
    for k in range(_NCHUNK):
        sl = slice(k * ec, (k + 1) * ec)
        outs.append(_tc_mlp(gathered[k], edges[sl],
                            w1ab, w1c, b1r, w2t, b2r))
    return jnp.concatenate(outs, axis=0)


# speedup vs baseline: 3.9048x; 1.1784x over previous
import functools

import jax
import jax.numpy as jnp
from jax import lax
from jax.experimental import pallas as pl
from jax.experimental.pallas import tpu as pltpu
from jax.experimental.pallas import tpu_sc as plsc

_NC, _NS = 2, 16
_NW = _NC * _NS
_CH = 40
_NRING = 2
_D_NODE = 128
_BE = 2560
_NCHUNK = 1


def _gather_body(nodes_hbm, s_hbm, r_hbm, out, shared, idx_s, idx_r,
                 bufs, sems_g, sems_w):
    e_total = s_hbm.shape[0]
    b_per_w = e_total // _NW
    nch = b_per_w // _CH
    wid = lax.axis_index("s") * _NC + lax.axis_index("c")
    base = wid * b_per_w

    n_nodes = nodes_hbm.shape[0]
    rows_per_tile = (n_nodes // _NS) // 8 * 8
    tail = n_nodes - _NS * rows_per_tile
    sid = lax.axis_index("s")
    pltpu.sync_copy(nodes_hbm.at[pl.ds(sid * rows_per_tile, rows_per_tile)],
                    shared.at[pl.ds(sid * rows_per_tile, rows_per_tile)])
    if tail:
        @pl.when(sid == _NS - 1)
        def _copy_tail():
            pltpu.sync_copy(nodes_hbm.at[pl.ds(_NS * rows_per_tile, tail)],
                            shared.at[pl.ds(_NS * rows_per_tile, tail)])
    pltpu.sync_copy(s_hbm.at[pl.ds(base, b_per_w)], idx_s)
    pltpu.sync_copy(r_hbm.at[pl.ds(base, b_per_w)], idx_r)
    plsc.subcore_barrier()
    idx = (idx_s, idx_r)

    def fire_gather(ring, c):
        return [
            pltpu.async_copy(
                shared.at[idx[st].at[pl.ds(c * _CH, _CH)]],
                bufs[ring][st], sems_g[ring][st])
            for st in (0, 1)
        ]

    def fire_write(ring, c):
        off = pl.multiple_of(base + c * _CH, 8)
        for st in (0, 1):
            pltpu.async_copy(
                bufs[ring][st],
                out.at[pl.ds(off, _CH), pl.ds(st * _D_NODE, _D_NODE)],
                sems_w[ring][st])

    def wait_write(ring):
        for st in (0, 1):
            pltpu.make_async_copy(
                bufs[ring][st],
                out.at[pl.ds(0, _CH), pl.ds(st * _D_NODE, _D_NODE)],
                sems_w[ring][st]).wait()

    ga = [fire_gather(j, j) for j in range(_NRING)]
    for j in range(_NRING):
        for d in ga[j]:
            d.wait()
        fire_write(j, j)

    def body(t, carry):
        c0 = t * _NRING
        ga = []
        for j in range(_NRING):
            wait_write(j)
            ga.append(fire_gather(j, c0 + j))
        for j in range(_NRING):
            for d in ga[j]:
                d.wait()
            fire_write(j, c0 + j)
        return carry

    lax.fori_loop(1, nch // _NRING, body, 0)
    for j in range(_NRING):
        wait_write(j)


def _sc_gather(nodes, senders, receivers):
    e_total = senders.shape[0]
    b_per_w = e_total // _NW
    mesh = plsc.VectorSubcoreMesh(core_axis_name="c", subcore_axis_name="s")
    f = pl.kernel(
        _gather_body,
        out_type=jax.ShapeDtypeStruct((e_total, 2 * _D_NODE), jnp.float32),
        mesh=mesh,
        scratch_types=[
            pltpu.VMEM_SHARED((nodes.shape[0], _D_NODE), jnp.float32),
            pltpu.VMEM((b_per_w,), jnp.int32),
            pltpu.VMEM((b_per_w,), jnp.int32),
            [[pltpu.VMEM((_CH, _D_NODE), jnp.float32) for _ in range(2)]
             for _ in range(_NRING)],
            [[pltpu.SemaphoreType.DMA for _ in range(2)] for _ in range(_NRING)],
            [[pltpu.SemaphoreType.DMA for _ in range(2)] for _ in range(_NRING)],
        ],
    )
    return f(nodes, senders, receivers)


def _mlp_body(g_ref, ef_ref, w1ab_ref, w1c_ref, b1_ref, w2_ref, b2_ref,
              out_ref):
    x = (jnp.dot(g_ref[...].astype(jnp.bfloat16), w1ab_ref[...],
                 preferred_element_type=jnp.float32)
         + jnp.dot(ef_ref[...].astype(jnp.bfloat16), w1c_ref[...],
                   preferred_element_type=jnp.float32)
         + b1_ref[...])
    h = 0.5 * jnp.tanh(0.5 * x) + 0.5
    out_ref[...] = (jnp.dot(h.astype(jnp.bfloat16), w2_ref[...],
                            preferred_element_type=jnp.float32)
                    + b2_ref[...])


def _tc_mlp(g, edges, w1ab, w1c, b1, w2, b2):
    e_total, d_edge = edges.shape
    d_hidden = w2.shape[0]
    d_out = w2.shape[1]
    grid = (e_total // _BE,)
    return pl.pallas_call(
        _mlp_body,
        grid=grid,
        in_specs=[
            pl.BlockSpec((_BE, 2 * _D_NODE), lambda i: (i, 0)),
            pl.BlockSpec((_BE, d_edge), lambda i: (i, 0)),
            pl.BlockSpec((2 * _D_NODE, d_hidden), lambda i: (0, 0)),
            pl.BlockSpec((d_edge, d_hidden), lambda i: (0, 0)),
            pl.BlockSpec((1, d_hidden), lambda i: (0, 0)),
            pl.BlockSpec((d_hidden, d_out), lambda i: (0, 0)),
            pl.BlockSpec((1, d_out), lambda i: (0, 0)),
        ],
        out_specs=pl.BlockSpec((_BE, d_out), lambda i: (i, 0)),
        out_shape=jax.ShapeDtypeStruct((e_total, d_out), jnp.float32),
    )(g, edges, w1ab, w1c, b1, w2, b2)


@jax.jit
def kernel(nodes, edges, W1, b1, W2, b2, senders, receivers):
    senders = senders.astype(jnp.int32)
    receivers = receivers.astype(jnp.int32)
    e_total = senders.shape[0]
    ec = e_total // _NCHUNK
    assert ec % (_NW * _NRING * _CH) == 0 and ec % _BE == 0

    d_node = nodes.shape[1]
    w1t = W1.T
    w1ab = w1t[:2 * d_node].astype(jnp.bfloat16)
    w1c = w1t[2 * d_node:].astype(jnp.bfloat16)
    b1r = b1.reshape(1, -1)
    w2t = W2.T.astype(jnp.bfloat16)
    b2r = b2.reshape(1, -1)

    gathered = []
    for k in range(_NCHUNK):
        sl = slice(k * ec, (k + 1) * ec)
        gathered.append(_sc_gather(nodes, senders[sl], receivers[sl]))
    outs = []
    for k in range(_NCHUNK):
        sl = slice(k * ec, (k + 1) * ec)
        outs.append(_tc_mlp(gathered[k], edges[sl],
                            w1ab, w1c, b1r, w2t, b2r))
    return jnp.concatenate(outs, axis=0)


# speedup vs baseline: 3.9727x; 1.0174x over previous
import functools

import jax
import jax.numpy as jnp
from jax import lax
from jax.experimental import pallas as pl
from jax.experimental.pallas import tpu as pltpu
from jax.experimental.pallas import tpu_sc as plsc

_NC, _NS = 2, 16
_NW = _NC * _NS
_CH = 40
_NRING = 2
_D_NODE = 128
_BE = 6400
_NCHUNK = 1


def _gather_body(nodes_hbm, s_hbm, r_hbm, out, shared, idx_s, idx_r,
                 bufs, sems_g, sems_w):
    e_total = s_hbm.shape[0]
    b_per_w = e_total // _NW
    nch = b_per_w // _CH
    wid = lax.axis_index("s") * _NC + lax.axis_index("c")
    base = wid * b_per_w

    n_nodes = nodes_hbm.shape[0]
    rows_per_tile = (n_nodes // _NS) // 8 * 8
    tail = n_nodes - _NS * rows_per_tile
    sid = lax.axis_index("s")
    pltpu.sync_copy(nodes_hbm.at[pl.ds(sid * rows_per_tile, rows_per_tile)],
                    shared.at[pl.ds(sid * rows_per_tile, rows_per_tile)])
    if tail:
        @pl.when(sid == _NS - 1)
        def _copy_tail():
            pltpu.sync_copy(nodes_hbm.at[pl.ds(_NS * rows_per_tile, tail)],
                            shared.at[pl.ds(_NS * rows_per_tile, tail)])
    pltpu.sync_copy(s_hbm.at[pl.ds(base, b_per_w)], idx_s)
    pltpu.sync_copy(r_hbm.at[pl.ds(base, b_per_w)], idx_r)
    plsc.subcore_barrier()
    idx = (idx_s, idx_r)

    def fire_gather(ring, c):
        return [
            pltpu.async_copy(
                shared.at[idx[st].at[pl.ds(c * _CH, _CH)]],
                bufs[ring][st], sems_g[ring][st])
            for st in (0, 1)
        ]

    def fire_write(ring, c):
        off = pl.multiple_of(base + c * _CH, 8)
        for st in (0, 1):
            pltpu.async_copy(
                bufs[ring][st],
                out.at[pl.ds(off, _CH), pl.ds(st * _D_NODE, _D_NODE)],
                sems_w[ring][st])

    def wait_write(ring):
        for st in (0, 1):
            pltpu.make_async_copy(
                bufs[ring][st],
                out.at[pl.ds(0, _CH), pl.ds(st * _D_NODE, _D_NODE)],
                sems_w[ring][st]).wait()

    ga = [fire_gather(j, j) for j in range(_NRING)]
    for j in range(_NRING):
        for d in ga[j]:
            d.wait()
        fire_write(j, j)

    def body(t, carry):
        c0 = t * _NRING
        ga = []
        for j in range(_NRING):
            wait_write(j)
            ga.append(fire_gather(j, c0 + j))
        for j in range(_NRING):
            for d in ga[j]:
                d.wait()
            fire_write(j, c0 + j)
        return carry

    lax.fori_loop(1, nch // _NRING, body, 0)
    for j in range(_NRING):
        wait_write(j)


def _sc_gather(nodes, senders, receivers):
    e_total = senders.shape[0]
    b_per_w = e_total // _NW
    mesh = plsc.VectorSubcoreMesh(core_axis_name="c", subcore_axis_name="s")
    f = pl.kernel(
        _gather_body,
        out_type=jax.ShapeDtypeStruct((e_total, 2 * _D_NODE), jnp.float32),
        mesh=mesh,
        scratch_types=[
            pltpu.VMEM_SHARED((nodes.shape[0], _D_NODE), jnp.float32),
            pltpu.VMEM((b_per_w,), jnp.int32),
            pltpu.VMEM((b_per_w,), jnp.int32),
            [[pltpu.VMEM((_CH, _D_NODE), jnp.float32) for _ in range(2)]
             for _ in range(_NRING)],
            [[pltpu.SemaphoreType.DMA for _ in range(2)] for _ in range(_NRING)],
            [[pltpu.SemaphoreType.DMA for _ in range(2)] for _ in range(_NRING)],
        ],
    )
    return f(nodes, senders, receivers)


def _mlp_body(g_ref, ef_ref, w1ab_ref, w1c_ref, b1_ref, w2_ref, b2_ref,
              out_ref):
    x = (jnp.dot(g_ref[...].astype(jnp.bfloat16), w1ab_ref[...],
                 preferred_element_type=jnp.float32)
         + jnp.dot(ef_ref[...].astype(jnp.bfloat16), w1c_ref[...],
                   preferred_element_type=jnp.float32)
         + b1_ref[...])
    h = 0.5 * jnp.tanh(0.5 * x) + 0.5
    out_ref[...] = (jnp.dot(h.astype(jnp.bfloat16), w2_ref[...],
                            preferred_element_type=jnp.float32)
                    + b2_ref[...])


def _tc_mlp(g, edges, w1ab, w1c, b1, w2, b2):
    e_total, d_edge = edges.shape
    d_hidden = w2.shape[0]
    d_out = w2.shape[1]
    grid = (e_total // _BE,)
    return pl.pallas_call(
        _mlp_body,
        grid=grid,
        in_specs=[
            pl.BlockSpec((_BE, 2 * _D_NODE), lambda i: (i, 0)),
            pl.BlockSpec((_BE, d_edge), lambda i: (i, 0)),
            pl.BlockSpec((2 * _D_NODE, d_hidden), lambda i: (0, 0)),
            pl.BlockSpec((d_edge, d_hidden), lambda i: (0, 0)),
            pl.BlockSpec((1, d_hidden), lambda i: (0, 0)),
            pl.BlockSpec((d_hidden, d_out), lambda i: (0, 0)),
            pl.BlockSpec((1, d_out), lambda i: (0, 0)),
        ],
        out_specs=pl.BlockSpec((_BE, d_out), lambda i: (i, 0)),
        out_shape=jax.ShapeDtypeStruct((e_total, d_out), jnp.float32),
    )(g, edges, w1ab, w1c, b1, w2, b2)


@jax.jit
def kernel(nodes, edges, W1, b1, W2, b2, senders, receivers):
    senders = senders.astype(jnp.int32)
    receivers = receivers.astype(jnp.int32)
    e_total = senders.shape[0]
    ec = e_total // _NCHUNK
    assert ec % (_NW * _NRING * _CH) == 0 and ec % _BE == 0

    d_node = nodes.shape[1]
    w1t = W1.T
    w1ab = w1t[:2 * d_node].astype(jnp.bfloat16)
    w1c = w1t[2 * d_node:].astype(jnp.bfloat16)
    b1r = b1.reshape(1, -1)
    w2t = W2.T.astype(jnp.bfloat16)
    b2r = b2.reshape(1, -1)

    gathered = []
    for k in range(_NCHUNK):
        sl = slice(k * ec, (k + 1) * ec)
        gathered.append(_sc_gather(nodes, senders[sl], receivers[sl]))
    outs = []
    for k in range(_NCHUNK):
        sl = slice(k * ec, (k + 1) * ec)
        outs.append(_tc_mlp(gathered[k], edges[sl],
                            w1ab, w1c, b1r, w2t, b2r))
    return jnp.concatenate(outs, axis=0)
